# Initial kernel scaffold; baseline (speedup 1.0000x reference)
import jax
import jax.numpy as jnp
from jax.experimental import pallas as pl


def kernel(x, edge_index, batch, params):
    raise NotImplementedError("write your pallas kernel here")


# speedup vs baseline: 22.4249x; 22.4249x over previous
import functools

import jax
import jax.numpy as jnp
from jax import lax
from jax.experimental import pallas as pl
from jax.experimental.pallas import tpu as pltpu
from jax.experimental.pallas import tpu_sc as plsc

N = 10000
E = 320000
G = 500
HID = 64

NC, NS = 2, 16
NW = NC * NS
CH = 128
EPT = 10240
EPAD = NW * EPT
NCHUNK = EPT // CH
NPAD = 10240
RPT = NPAD // NS

_f32 = jnp.float32


def _sc_mesh():
    return plsc.VectorSubcoreMesh(
        core_axis_name="c", subcore_axis_name="s", num_cores=NC, num_subcores=NS)


@functools.cache
def _get_sc_deg():
    @functools.partial(
        pl.kernel,
        out_type=jax.ShapeDtypeStruct((NC, NPAD, 16), _f32),
        mesh=_sc_mesh(),
        scratch_types=[
            pltpu.VMEM((CH,), jnp.int32),
            pltpu.VMEM((CH, 16), _f32),
            pltpu.VMEM_SHARED((NPAD, 16), _f32),
        ],
        compiler_params=pltpu.CompilerParams(use_tc_tiling_on_sc=False),
    )
    def _sc_deg(ones_hbm, dst_hbm, out_hbm, didx, ones_v, acc):
        c = lax.axis_index("c")
        s = lax.axis_index("s")
        pltpu.sync_copy(ones_hbm.at[pl.ds(0, CH)], ones_v)
        pltpu.sync_copy(ones_hbm.at[pl.ds(s * RPT, RPT)],
                        acc.at[pl.ds(s * RPT, RPT)])
        plsc.subcore_barrier()
        base = (c * NS + s) * EPT

        def body(i, carry):
            off = base + i * CH
            pltpu.sync_copy(dst_hbm.at[pl.ds(off, CH)], didx)
            pltpu.sync_copy(ones_v, acc.at[didx], add=True)
            return carry

        lax.fori_loop(0, NCHUNK, body, 0)
        plsc.subcore_barrier()
        pltpu.sync_copy(acc.at[pl.ds(s * RPT, RPT)],
                        out_hbm.at[c, pl.ds(s * RPT, RPT)])

    return _sc_deg


@functools.cache
def _get_sc_agg(F):
    @functools.partial(
        pl.kernel,
        out_type=jax.ShapeDtypeStruct((NC, NPAD, F), _f32),
        mesh=_sc_mesh(),
        scratch_types=[
            pltpu.VMEM((CH,), jnp.int32),
            pltpu.VMEM((CH,), jnp.int32),
            pltpu.VMEM((CH, F), _f32),
            pltpu.VMEM_SHARED((NPAD, F), _f32),
            pltpu.SemaphoreType.DMA,
        ],
        compiler_params=pltpu.CompilerParams(use_tc_tiling_on_sc=False),
    )
    def agg(table_hbm, src_hbm, dst_hbm, out_hbm, sidx, didx, rows, acc, sem):
        c = lax.axis_index("c")
        s = lax.axis_index("s")
        pltpu.sync_copy(table_hbm.at[pl.ds(s * RPT, RPT)],
                        acc.at[pl.ds(s * RPT, RPT)])
        plsc.subcore_barrier()
        base = (c * NS + s) * EPT

        def body(i, carry):
            off = base + i * CH
            pltpu.sync_copy(src_hbm.at[pl.ds(off, CH)], sidx)
            pltpu.sync_copy(dst_hbm.at[pl.ds(off, CH)], didx)
            pltpu.async_copy(table_hbm.at[sidx], rows, sem).wait()
            pltpu.sync_copy(rows, acc.at[didx], add=True)
            return carry

        lax.fori_loop(0, NCHUNK, body, 0)
        plsc.subcore_barrier()
        pltpu.sync_copy(acc.at[pl.ds(s * RPT, RPT)], out_hbm.at[c, pl.ds(s * RPT, RPT)])

    return agg


def _dinv_from(dp):
    deg = dp[0, :N, 0:1] + dp[1, :N, 0:1] - 1.0
    return lax.rsqrt(deg)


def _padN(t):
    return jnp.concatenate([t, jnp.zeros((NPAD - N, t.shape[1]), _f32)], axis=0)


def _tc1_body(dp_ref, x_ref, w1_ref, h1p_ref):
    dinv = _dinv_from(dp_ref[...])
    h = jnp.dot(x_ref[...], w1_ref[...], preferred_element_type=_f32)
    h1p_ref[...] = _padN(h * dinv)


def _tc1(dp, x, w1):
    return pl.pallas_call(
        _tc1_body,
        out_shape=jax.ShapeDtypeStruct((NPAD, 32), _f32),
    )(dp, x, w1)


def _tc_epi_body(dp_ref, a_ref, hp_ref, b_ref, w_ref, x_ref, hn_ref):
    dinv = _dinv_from(dp_ref[...])
    a = a_ref[...]
    g = a[0, :N] + a[1, :N] - hp_ref[...][:N]
    xo = jax.nn.relu(dinv * g + b_ref[...])
    x_ref[...] = xo
    hn_ref[...] = _padN(jnp.dot(xo, w_ref[...], preferred_element_type=_f32) * dinv)


def _tc_epi(dp, a, hp, b, w):
    return pl.pallas_call(
        _tc_epi_body,
        out_shape=(
            jax.ShapeDtypeStruct((N, 32), _f32),
            jax.ShapeDtypeStruct((NPAD, 32), _f32),
        ),
    )(dp, a, hp, b, w)


def _tc4_body(dp_ref, a_ref, hp_ref, x1_ref, x2_ref, b3_ref, wg_ref, bg_ref,
              wk_ref, wv_ref, out_ref):
    dinv = _dinv_from(dp_ref[...])
    a = a_ref[...]
    g = a[0, :N] + a[1, :N] - hp_ref[...][:N]
    x3 = jax.nn.relu(dinv * g + b3_ref[...])
    xc = jnp.concatenate([x1_ref[...], x2_ref[...], x3], axis=1)
    h = jnp.dot(xc, wg_ref[...], preferred_element_type=_f32) + bg_ref[...]
    hk = jnp.dot(h, wk_ref[...], preferred_element_type=_f32) * dinv
    hv = jnp.dot(h, wv_ref[...], preferred_element_type=_f32) * dinv
    out_ref[...] = _padN(jnp.concatenate([hk, hv], axis=1))


def _tc4(dp, a3, h3p, x1, x2, b3, wg, bg, wk, wv):
    return pl.pallas_call(
        _tc4_body,
        out_shape=jax.ShapeDtypeStruct((NPAD, 128), _f32),
    )(dp, a3, h3p, x1, x2, b3, wg, bg, wk, wv)


def _tc5_body(dp_ref, a_ref, hp_ref, bk_ref, bv_ref, sg_ref, wq_ref, bq_ref,
              sc_ref, v_ref, m_ref):
    dinv = _dinv_from(dp_ref[...])
    a = a_ref[...]
    g = a[0, :N] + a[1, :N] - hp_ref[...][:N]
    Km = dinv * g[:, :64] + bk_ref[...]
    Vm = dinv * g[:, 64:] + bv_ref[...]
    Qp = jnp.dot(sg_ref[...], wq_ref[...], preferred_element_type=_f32) + bq_ref[...]
    Qp4 = jnp.concatenate([Qp, Qp, Qp, Qp], axis=0)
    hc = lax.broadcasted_iota(jnp.int32, (16, 64), 0) // 4
    hd = lax.broadcasted_iota(jnp.int32, (16, 64), 1) // 16
    qmask = Qp4 * (hc == hd).astype(_f32) * 0.125
    scores = lax.dot_general(Km, qmask, (((1,), (1,)), ((), ())),
                             preferred_element_type=_f32)
    sc_ref[...] = scores
    v_ref[...] = Vm
    m_ref[...] = jnp.max(scores, axis=0, keepdims=True)


def _tc5(dp, akv, hkvp, bk, bv, sg, wq, bq):
    return pl.pallas_call(
        _tc5_body,
        out_shape=(
            jax.ShapeDtypeStruct((N, 16), _f32),
            jax.ShapeDtypeStruct((N, 64), _f32),
            jax.ShapeDtypeStruct((1, 16), _f32),
        ),
    )(dp, akv, hkvp, bk, bv, sg, wq, bq)


NBK = 1000
NB = N // NBK


def _tc6_body(sc_ref, v_ref, m_ref, b_ref, out_ref):
    i = pl.program_id(0)

    @pl.when(i == 0)
    def _():
        out_ref[...] = jnp.zeros_like(out_ref)

    Ex = jnp.exp(sc_ref[...] - m_ref[...])
    V = v_ref[...]
    pieces = [Ex]
    for q in range(4):
        for hb in range(4):
            c = hb * 4 + q
            pieces.append(Ex[:, c:c + 1] * V[:, hb * 16:(hb + 1) * 16])
    EP = jnp.concatenate(pieces, axis=1)
    gids = lax.broadcasted_iota(jnp.int32, (G, NBK), 0)
    onehot = (gids == b_ref[0]).astype(_f32)
    out_ref[...] += jnp.dot(onehot, EP, preferred_element_type=_f32)


def _tc6(scores, V, m, batch3):
    return pl.pallas_call(
        _tc6_body,
        grid=(NB,),
        in_specs=[
            pl.BlockSpec((NBK, 16), lambda i: (i, 0)),
            pl.BlockSpec((NBK, 64), lambda i: (i, 0)),
            pl.BlockSpec((1, 16), lambda i: (0, 0)),
            pl.BlockSpec((1, 1, NBK), lambda i: (i, 0, 0)),
        ],
        out_specs=pl.BlockSpec((G, 272), lambda i: (0, 0)),
        out_shape=jax.ShapeDtypeStruct((G, 272), _f32),
    )(scores, V, m, batch3)


def _tc7_body(ss_ref, sg_ref, wqg_ref, bqg_ref, wog_ref, bog_ref,
              wqs_ref, bqs_ref, wks_ref, bks_ref, wvs_ref, bvs_ref,
              wos_ref, bos_ref, si_ref, wqi_ref, bqi_ref, wki_ref, bki_ref,
              wvi_ref, bvi_ref, woi_ref, boi_ref, wl2_ref, bl2_ref,
              wla_ref, bla_ref, wlb_ref, blb_ref, logits_ref, feat_ref):
    SS = ss_ref[...]
    Qp = jnp.dot(sg_ref[...], wqg_ref[...], preferred_element_type=_f32) + bqg_ref[...]
    D16 = SS[:, :16]
    pieces = []
    for q in range(4):
        for hb in range(4):
            c = hb * 4 + q
            den = D16[:, c:c + 1]
            num = SS[:, 16 + q * 64 + hb * 16: 16 + q * 64 + (hb + 1) * 16]
            att = jnp.where(den > 0, num / den, 0.0)
            pieces.append(att + Qp[q:q + 1, hb * 16:(hb + 1) * 16])
    X = [jnp.concatenate(pieces[q * 4:(q + 1) * 4], axis=1) for q in range(4)]
    wog, bog = wog_ref[...], bog_ref[...]
    X = [x + jax.nn.relu(jnp.dot(x, wog, preferred_element_type=_f32) + bog)
         for x in X]
    wqs, bqs = wqs_ref[...], bqs_ref[...]
    wks, bks = wks_ref[...], bks_ref[...]
    wvs, bvs = wvs_ref[...], bvs_ref[...]
    Q2 = [jnp.dot(t, wqs, preferred_element_type=_f32) + bqs for t in X]
    K2 = [jnp.dot(t, wks, preferred_element_type=_f32) + bks for t in X]
    V2 = [jnp.dot(t, wvs, preferred_element_type=_f32) + bvs for t in X]
    O = []
    for q in range(4):
        outq = []
        for h in range(4):
            sl = slice(h * 16, (h + 1) * 16)
            sv = [jnp.sum(Q2[q][:, sl] * K2[k][:, sl], axis=1, keepdims=True) * 0.125
                  for k in range(4)]
            mx = jnp.maximum(jnp.maximum(sv[0], sv[1]), jnp.maximum(sv[2], sv[3]))
            ev = [jnp.exp(t - mx) for t in sv]
            Z = ev[0] + ev[1] + ev[2] + ev[3]
            att = sum(ev[k] / Z * V2[k][:, sl] for k in range(4))
            outq.append(Q2[q][:, sl] + att)
        O.append(jnp.concatenate(outq, axis=1))
    wos, bos = wos_ref[...], bos_ref[...]
    X2 = [o + jax.nn.relu(jnp.dot(o, wos, preferred_element_type=_f32) + bos)
          for o in O]
    Qp3 = jnp.dot(si_ref[...], wqi_ref[...], preferred_element_type=_f32) + bqi_ref[...]
    wki, bki = wki_ref[...], bki_ref[...]
    wvi, bvi = wvi_ref[...], bvi_ref[...]
    K3 = [jnp.dot(t, wki, preferred_element_type=_f32) + bki for t in X2]
    V3 = [jnp.dot(t, wvi, preferred_element_type=_f32) + bvi for t in X2]
    outh = []
    for h in range(4):
        sl = slice(h * 16, (h + 1) * 16)
        sv = [jnp.sum(K3[k][:, sl] * Qp3[:, sl], axis=1, keepdims=True) * 0.125
              for k in range(4)]
        mx = jnp.maximum(jnp.maximum(sv[0], sv[1]), jnp.maximum(sv[2], sv[3]))
        ev = [jnp.exp(t - mx) for t in sv]
        Z = ev[0] + ev[1] + ev[2] + ev[3]
        att = sum(ev[k] / Z * V3[k][:, sl] for k in range(4))
        outh.append(Qp3[:, sl] + att)
    O3 = jnp.concatenate(outh, axis=1)
    X3 = O3 + jax.nn.relu(jnp.dot(O3, woi_ref[...], preferred_element_type=_f32)
                          + boi_ref[...])
    pooled = jnp.dot(X3, wl2_ref[...], preferred_element_type=_f32) + bl2_ref[...]
    feat = jax.nn.relu(jnp.dot(pooled, wla_ref[...], preferred_element_type=_f32)
                       + bla_ref[...])
    logits = jnp.dot(feat, wlb_ref[...], preferred_element_type=_f32) + blb_ref[...]
    logits_ref[...] = logits
    feat_ref[...] = feat


def _tc7(args):
    return pl.pallas_call(
        _tc7_body,
        out_shape=(
            jax.ShapeDtypeStruct((G, 2), _f32),
            jax.ShapeDtypeStruct((G, 16), _f32),
        ),
    )(*args)


def _row(b):
    return b.reshape(1, -1)


@jax.jit
def kernel(x, edge_index, batch, params):
    src = edge_index[0].astype(jnp.int32)
    dst = edge_index[1].astype(jnp.int32)
    pad = EPAD - E
    srcp = jnp.concatenate([src, jnp.full((pad,), N, jnp.int32)])
    dstp = jnp.concatenate([dst, jnp.full((pad,), N, jnp.int32)])
    ones16 = jnp.ones((NPAD, 16), _f32)

    p = params
    dp = _get_sc_deg()(ones16, dstp)

    h1p = _tc1(dp, x, p['conv1']['w'])
    a1 = _get_sc_agg(32)(h1p, srcp, dstp)
    x1, h2p = _tc_epi(dp, a1, h1p, _row(p['conv1']['b']), p['conv2']['w'])
    a2 = _get_sc_agg(32)(h2p, srcp, dstp)
    x2, h3p = _tc_epi(dp, a2, h2p, _row(p['conv2']['b']), p['conv3']['w'])
    a3 = _get_sc_agg(32)(h3p, srcp, dstp)

    pg = p['pma_g']
    hkvp = _tc4(dp, a3, h3p, x1, x2, _row(p['conv3']['b']),
                p['gmt_lin1']['w'], _row(p['gmt_lin1']['b']),
                pg['conv_k']['w'], pg['conv_v']['w'])
    akv = _get_sc_agg(128)(hkvp, srcp, dstp)
    scores, V, m = _tc5(dp, akv, hkvp, _row(pg['conv_k']['b']),
                        _row(pg['conv_v']['b']), pg['S'][0],
                        pg['fc_q']['w'], _row(pg['fc_q']['b']))

    batch3 = batch.astype(jnp.int32).reshape(NB, 1, NBK)
    SS = _tc6(scores, V, m, batch3)

    ps, pi = p['sab'], p['pma_i']
    logits, feat = _tc7((
        SS, pg['S'][0], pg['fc_q']['w'], _row(pg['fc_q']['b']),
        pg['fc_o']['w'], _row(pg['fc_o']['b']),
        ps['fc_q']['w'], _row(ps['fc_q']['b']),
        ps['lin_k']['w'], _row(ps['lin_k']['b']),
        ps['lin_v']['w'], _row(ps['lin_v']['b']),
        ps['fc_o']['w'], _row(ps['fc_o']['b']),
        pi['S'][0], pi['fc_q']['w'], _row(pi['fc_q']['b']),
        pi['lin_k']['w'], _row(pi['lin_k']['b']),
        pi['lin_v']['w'], _row(pi['lin_v']['b']),
        pi['fc_o']['w'], _row(pi['fc_o']['b']),
        p['gmt_lin2']['w'], _row(p['gmt_lin2']['b']),
        p['lin1']['w'], _row(p['lin1']['b']),
        p['lin2']['w'], _row(p['lin2']['b']),
    ))
    return logits, feat


# speedup vs baseline: 41.4769x; 1.8496x over previous
import functools

import jax
import jax.numpy as jnp
from jax import lax
from jax.experimental import pallas as pl
from jax.experimental.pallas import tpu as pltpu
from jax.experimental.pallas import tpu_sc as plsc

N = 10000
E = 320000
G = 500
HID = 64

NC, NS = 2, 16
NW = NC * NS
CH = 128
EPT = 10240
EPAD = NW * EPT
NCHUNK = EPT // CH
NPAD = 10240
RPT = NPAD // NS

_f32 = jnp.float32


def _sc_mesh():
    return plsc.VectorSubcoreMesh(
        core_axis_name="c", subcore_axis_name="s", num_cores=NC, num_subcores=NS)


@functools.cache
def _get_sc_deg():
    @functools.partial(
        pl.kernel,
        out_type=jax.ShapeDtypeStruct((NC, NPAD, 16), _f32),
        mesh=_sc_mesh(),
        scratch_types=[
            pltpu.VMEM((NCHUNK, CH), jnp.int32),
            pltpu.VMEM((CH, 16), _f32),
            pltpu.VMEM_SHARED((NPAD, 16), _f32),
            pltpu.SemaphoreType.DMA,
        ],
        compiler_params=pltpu.CompilerParams(use_tc_tiling_on_sc=False),
    )
    def _sc_deg(ones_hbm, dst_hbm, out_hbm, didx, ones_v, acc, ssem):
        c = lax.axis_index("c")
        s = lax.axis_index("s")
        wid = c * NS + s
        pltpu.sync_copy(dst_hbm.at[wid], didx)
        pltpu.sync_copy(ones_hbm.at[pl.ds(0, CH)], ones_v)
        pltpu.sync_copy(ones_hbm.at[pl.ds(s * RPT, RPT)],
                        acc.at[pl.ds(s * RPT, RPT)])
        plsc.subcore_barrier()

        @pl.loop(0, NCHUNK)
        def _(i):
            pltpu.async_copy(ones_v, acc.at[didx.at[i]], ssem, add=True)

        @pl.loop(0, NCHUNK)
        def _(i):
            pltpu.make_async_copy(ones_v, acc.at[didx.at[i]], ssem).wait()

        plsc.subcore_barrier()
        pltpu.sync_copy(acc.at[pl.ds(s * RPT, RPT)],
                        out_hbm.at[c, pl.ds(s * RPT, RPT)])

    return _sc_deg


NBUF = 4


@functools.cache
def _get_sc_agg(F):
    @functools.partial(
        pl.kernel,
        out_type=jax.ShapeDtypeStruct((NC, NPAD, F), _f32),
        mesh=_sc_mesh(),
        scratch_types=[
            pltpu.VMEM((NCHUNK, CH), jnp.int32),
            pltpu.VMEM((NCHUNK, CH), jnp.int32),
            pltpu.VMEM((NBUF, CH, F), _f32),
            pltpu.VMEM_SHARED((NPAD, F), _f32),
            pltpu.SemaphoreType.DMA((NBUF,)),
            pltpu.SemaphoreType.DMA((NBUF,)),
        ],
        compiler_params=pltpu.CompilerParams(use_tc_tiling_on_sc=False),
    )
    def agg(table_hbm, src_hbm, dst_hbm, out_hbm, sidx, didx, rows, acc,
            gsem, ssem):
        c = lax.axis_index("c")
        s = lax.axis_index("s")
        wid = c * NS + s
        pltpu.sync_copy(src_hbm.at[wid], sidx)
        pltpu.sync_copy(dst_hbm.at[wid], didx)
        pltpu.sync_copy(table_hbm.at[pl.ds(s * RPT, RPT)],
                        acc.at[pl.ds(s * RPT, RPT)])
        plsc.subcore_barrier()

        for b in range(NBUF):
            pltpu.async_copy(table_hbm.at[sidx.at[b]], rows.at[b], gsem.at[b])

        @pl.loop(0, NCHUNK, step=NBUF)
        def _(i0):
            for b in range(NBUF):
                i = i0 + b
                pltpu.make_async_copy(table_hbm.at[sidx.at[i]], rows.at[b],
                                      gsem.at[b]).wait()
                pltpu.async_copy(rows.at[b], acc.at[didx.at[i]], ssem.at[b],
                                 add=True)

                @pl.when(i + NBUF < NCHUNK)
                def _():
                    pltpu.make_async_copy(rows.at[b], acc.at[didx.at[i]],
                                          ssem.at[b]).wait()
                    pltpu.async_copy(table_hbm.at[sidx.at[i + NBUF]],
                                     rows.at[b], gsem.at[b])

        for b in range(NBUF):
            pltpu.make_async_copy(rows.at[b], acc.at[didx.at[0]],
                                  ssem.at[b]).wait()

        plsc.subcore_barrier()
        pltpu.sync_copy(acc.at[pl.ds(s * RPT, RPT)], out_hbm.at[c, pl.ds(s * RPT, RPT)])

    return agg


NCHUNK_KV = EPAD // NS // CH


@functools.cache
def _get_sc_agg_kv():
    @functools.partial(
        pl.kernel,
        out_type=jax.ShapeDtypeStruct((NC, NPAD, 64), _f32),
        mesh=_sc_mesh(),
        scratch_types=[
            pltpu.VMEM((NCHUNK_KV, CH), jnp.int32),
            pltpu.VMEM((NCHUNK_KV, CH), jnp.int32),
            pltpu.VMEM((NBUF, CH, 64), _f32),
            pltpu.VMEM_SHARED((NPAD, 64), _f32),
            pltpu.SemaphoreType.DMA((NBUF,)),
            pltpu.SemaphoreType.DMA((NBUF,)),
        ],
        compiler_params=pltpu.CompilerParams(use_tc_tiling_on_sc=False),
    )
    def agg_kv(hk_hbm, hv_hbm, src_hbm, dst_hbm, out_hbm, sidx, didx, rows,
               acc, gsem, ssem):
        c = lax.axis_index("c")
        s = lax.axis_index("s")
        pltpu.sync_copy(src_hbm.at[s], sidx)
        pltpu.sync_copy(dst_hbm.at[s], didx)

        def run(table_hbm):
            pltpu.sync_copy(table_hbm.at[pl.ds(s * RPT, RPT)],
                            acc.at[pl.ds(s * RPT, RPT)])
            plsc.subcore_barrier()

            for b in range(NBUF):
                pltpu.async_copy(table_hbm.at[sidx.at[b]], rows.at[b],
                                 gsem.at[b])

            @pl.loop(0, NCHUNK_KV, step=NBUF)
            def _(i0):
                for b in range(NBUF):
                    i = i0 + b
                    pltpu.make_async_copy(table_hbm.at[sidx.at[i]],
                                          rows.at[b], gsem.at[b]).wait()
                    pltpu.async_copy(rows.at[b], acc.at[didx.at[i]],
                                     ssem.at[b], add=True)

                    @pl.when(i + NBUF < NCHUNK_KV)
                    def _():
                        pltpu.make_async_copy(rows.at[b], acc.at[didx.at[i]],
                                              ssem.at[b]).wait()
                        pltpu.async_copy(table_hbm.at[sidx.at[i + NBUF]],
                                         rows.at[b], gsem.at[b])

            for b in range(NBUF):
                pltpu.make_async_copy(rows.at[b], acc.at[didx.at[0]],
                                      ssem.at[b]).wait()

            plsc.subcore_barrier()
            pltpu.sync_copy(acc.at[pl.ds(s * RPT, RPT)],
                            out_hbm.at[c, pl.ds(s * RPT, RPT)])

        @pl.when(c == 0)
        def _():
            run(hk_hbm)

        @pl.when(c == 1)
        def _():
            run(hv_hbm)

    return agg_kv


def _dinv_from(dp):
    deg = dp[0, :N, 0:1] + dp[1, :N, 0:1] - 1.0
    return lax.rsqrt(deg)


def _padN(t):
    return jnp.concatenate([t, jnp.zeros((NPAD - N, t.shape[1]), _f32)], axis=0)


def _tc1_body(dp_ref, x_ref, w1_ref, h1p_ref):
    dinv = _dinv_from(dp_ref[...])
    h = jnp.dot(x_ref[...], w1_ref[...], preferred_element_type=_f32)
    h1p_ref[...] = _padN(h * dinv)


def _tc1(dp, x, w1):
    return pl.pallas_call(
        _tc1_body,
        out_shape=jax.ShapeDtypeStruct((NPAD, 32), _f32),
    )(dp, x, w1)


def _tc_epi_body(dp_ref, a_ref, hp_ref, b_ref, w_ref, x_ref, hn_ref):
    dinv = _dinv_from(dp_ref[...])
    a = a_ref[...]
    g = a[0, :N] + a[1, :N] - hp_ref[...][:N]
    xo = jax.nn.relu(dinv * g + b_ref[...])
    x_ref[...] = xo
    hn_ref[...] = _padN(jnp.dot(xo, w_ref[...], preferred_element_type=_f32) * dinv)


def _tc_epi(dp, a, hp, b, w):
    return pl.pallas_call(
        _tc_epi_body,
        out_shape=(
            jax.ShapeDtypeStruct((N, 32), _f32),
            jax.ShapeDtypeStruct((NPAD, 32), _f32),
        ),
    )(dp, a, hp, b, w)


def _tc4_body(dp_ref, a_ref, hp_ref, x1_ref, x2_ref, b3_ref, wg_ref, bg_ref,
              wk_ref, wv_ref, hk_ref, hv_ref):
    dinv = _dinv_from(dp_ref[...])
    a = a_ref[...]
    g = a[0, :N] + a[1, :N] - hp_ref[...][:N]
    x3 = jax.nn.relu(dinv * g + b3_ref[...])
    xc = jnp.concatenate([x1_ref[...], x2_ref[...], x3], axis=1)
    h = jnp.dot(xc, wg_ref[...], preferred_element_type=_f32) + bg_ref[...]
    hk = jnp.dot(h, wk_ref[...], preferred_element_type=_f32) * dinv
    hv = jnp.dot(h, wv_ref[...], preferred_element_type=_f32) * dinv
    hk_ref[...] = _padN(hk)
    hv_ref[...] = _padN(hv)


def _tc4(dp, a3, h3p, x1, x2, b3, wg, bg, wk, wv):
    return pl.pallas_call(
        _tc4_body,
        out_shape=(
            jax.ShapeDtypeStruct((NPAD, 64), _f32),
            jax.ShapeDtypeStruct((NPAD, 64), _f32),
        ),
    )(dp, a3, h3p, x1, x2, b3, wg, bg, wk, wv)


def _tc5_body(dp_ref, a_ref, bk_ref, bv_ref, sg_ref, wq_ref,
              bq_ref, sc_ref, v_ref, m_ref):
    dinv = _dinv_from(dp_ref[...])
    a = a_ref[...]
    Km = dinv * a[0, :N] + bk_ref[...]
    Vm = dinv * a[1, :N] + bv_ref[...]
    Qp = jnp.dot(sg_ref[...], wq_ref[...], preferred_element_type=_f32) + bq_ref[...]
    Qp4 = jnp.concatenate([Qp, Qp, Qp, Qp], axis=0)
    hc = lax.broadcasted_iota(jnp.int32, (16, 64), 0) // 4
    hd = lax.broadcasted_iota(jnp.int32, (16, 64), 1) // 16
    qmask = Qp4 * (hc == hd).astype(_f32) * 0.125
    scores = lax.dot_general(Km, qmask, (((1,), (1,)), ((), ())),
                             preferred_element_type=_f32)
    sc_ref[...] = scores
    v_ref[...] = Vm
    m_ref[...] = jnp.max(scores, axis=0, keepdims=True)


def _tc5(dp, akv, bk, bv, sg, wq, bq):
    return pl.pallas_call(
        _tc5_body,
        out_shape=(
            jax.ShapeDtypeStruct((N, 16), _f32),
            jax.ShapeDtypeStruct((N, 64), _f32),
            jax.ShapeDtypeStruct((1, 16), _f32),
        ),
    )(dp, akv, bk, bv, sg, wq, bq)


NBK = 1000
NB = N // NBK


def _tc6_body(sc_ref, v_ref, m_ref, b_ref, out_ref):
    i = pl.program_id(0)

    @pl.when(i == 0)
    def _():
        out_ref[...] = jnp.zeros_like(out_ref)

    Ex = jnp.exp(sc_ref[...] - m_ref[...])
    V = v_ref[...]
    pieces = [Ex]
    for q in range(4):
        for hb in range(4):
            c = hb * 4 + q
            pieces.append(Ex[:, c:c + 1] * V[:, hb * 16:(hb + 1) * 16])
    EP = jnp.concatenate(pieces, axis=1)
    gids = lax.broadcasted_iota(jnp.int32, (G, NBK), 0)
    onehot = (gids == b_ref[0]).astype(_f32)
    out_ref[...] += jnp.dot(onehot, EP, preferred_element_type=_f32)


def _tc6(scores, V, m, batch3):
    return pl.pallas_call(
        _tc6_body,
        grid=(NB,),
        in_specs=[
            pl.BlockSpec((NBK, 16), lambda i: (i, 0)),
            pl.BlockSpec((NBK, 64), lambda i: (i, 0)),
            pl.BlockSpec((1, 16), lambda i: (0, 0)),
            pl.BlockSpec((1, 1, NBK), lambda i: (i, 0, 0)),
        ],
        out_specs=pl.BlockSpec((G, 272), lambda i: (0, 0)),
        out_shape=jax.ShapeDtypeStruct((G, 272), _f32),
    )(scores, V, m, batch3)


def _tc7_body(ss_ref, sg_ref, wqg_ref, bqg_ref, wog_ref, bog_ref,
              wqs_ref, bqs_ref, wks_ref, bks_ref, wvs_ref, bvs_ref,
              wos_ref, bos_ref, si_ref, wqi_ref, bqi_ref, wki_ref, bki_ref,
              wvi_ref, bvi_ref, woi_ref, boi_ref, wl2_ref, bl2_ref,
              wla_ref, bla_ref, wlb_ref, blb_ref, logits_ref, feat_ref):
    SS = ss_ref[...]
    Qp = jnp.dot(sg_ref[...], wqg_ref[...], preferred_element_type=_f32) + bqg_ref[...]
    D16 = SS[:, :16]
    pieces = []
    for q in range(4):
        for hb in range(4):
            c = hb * 4 + q
            den = D16[:, c:c + 1]
            num = SS[:, 16 + q * 64 + hb * 16: 16 + q * 64 + (hb + 1) * 16]
            att = jnp.where(den > 0, num / den, 0.0)
            pieces.append(att + Qp[q:q + 1, hb * 16:(hb + 1) * 16])
    X = [jnp.concatenate(pieces[q * 4:(q + 1) * 4], axis=1) for q in range(4)]
    wog, bog = wog_ref[...], bog_ref[...]
    X = [x + jax.nn.relu(jnp.dot(x, wog, preferred_element_type=_f32) + bog)
         for x in X]
    wqs, bqs = wqs_ref[...], bqs_ref[...]
    wks, bks = wks_ref[...], bks_ref[...]
    wvs, bvs = wvs_ref[...], bvs_ref[...]
    Q2 = [jnp.dot(t, wqs, preferred_element_type=_f32) + bqs for t in X]
    K2 = [jnp.dot(t, wks, preferred_element_type=_f32) + bks for t in X]
    V2 = [jnp.dot(t, wvs, preferred_element_type=_f32) + bvs for t in X]
    O = []
    for q in range(4):
        outq = []
        for h in range(4):
            sl = slice(h * 16, (h + 1) * 16)
            sv = [jnp.sum(Q2[q][:, sl] * K2[k][:, sl], axis=1, keepdims=True) * 0.125
                  for k in range(4)]
            mx = jnp.maximum(jnp.maximum(sv[0], sv[1]), jnp.maximum(sv[2], sv[3]))
            ev = [jnp.exp(t - mx) for t in sv]
            Z = ev[0] + ev[1] + ev[2] + ev[3]
            att = sum(ev[k] / Z * V2[k][:, sl] for k in range(4))
            outq.append(Q2[q][:, sl] + att)
        O.append(jnp.concatenate(outq, axis=1))
    wos, bos = wos_ref[...], bos_ref[...]
    X2 = [o + jax.nn.relu(jnp.dot(o, wos, preferred_element_type=_f32) + bos)
          for o in O]
    Qp3 = jnp.dot(si_ref[...], wqi_ref[...], preferred_element_type=_f32) + bqi_ref[...]
    wki, bki = wki_ref[...], bki_ref[...]
    wvi, bvi = wvi_ref[...], bvi_ref[...]
    K3 = [jnp.dot(t, wki, preferred_element_type=_f32) + bki for t in X2]
    V3 = [jnp.dot(t, wvi, preferred_element_type=_f32) + bvi for t in X2]
    outh = []
    for h in range(4):
        sl = slice(h * 16, (h + 1) * 16)
        sv = [jnp.sum(K3[k][:, sl] * Qp3[:, sl], axis=1, keepdims=True) * 0.125
              for k in range(4)]
        mx = jnp.maximum(jnp.maximum(sv[0], sv[1]), jnp.maximum(sv[2], sv[3]))
        ev = [jnp.exp(t - mx) for t in sv]
        Z = ev[0] + ev[1] + ev[2] + ev[3]
        att = sum(ev[k] / Z * V3[k][:, sl] for k in range(4))
        outh.append(Qp3[:, sl] + att)
    O3 = jnp.concatenate(outh, axis=1)
    X3 = O3 + jax.nn.relu(jnp.dot(O3, woi_ref[...], preferred_element_type=_f32)
                          + boi_ref[...])
    pooled = jnp.dot(X3, wl2_ref[...], preferred_element_type=_f32) + bl2_ref[...]
    feat = jax.nn.relu(jnp.dot(pooled, wla_ref[...], preferred_element_type=_f32)
                       + bla_ref[...])
    logits = jnp.dot(feat, wlb_ref[...], preferred_element_type=_f32) + blb_ref[...]
    logits_ref[...] = logits
    feat_ref[...] = feat


def _tc7(args):
    return pl.pallas_call(
        _tc7_body,
        out_shape=(
            jax.ShapeDtypeStruct((G, 2), _f32),
            jax.ShapeDtypeStruct((G, 16), _f32),
        ),
    )(*args)


def _row(b):
    return b.reshape(1, -1)


@jax.jit
def kernel(x, edge_index, batch, params):
    src = edge_index[0].astype(jnp.int32)
    dst = edge_index[1].astype(jnp.int32)
    pad = EPAD - E
    srcp = jnp.concatenate([src, jnp.full((pad,), N, jnp.int32)])
    dstp = jnp.concatenate([dst, jnp.full((pad,), N, jnp.int32)])
    srcp = srcp.reshape(NW, NCHUNK, CH)
    dstp = dstp.reshape(NW, NCHUNK, CH)
    ones16 = jnp.ones((NPAD, 16), _f32)

    p = params
    dp = _get_sc_deg()(ones16, dstp)

    h1p = _tc1(dp, x, p['conv1']['w'])
    a1 = _get_sc_agg(32)(h1p, srcp, dstp)
    x1, h2p = _tc_epi(dp, a1, h1p, _row(p['conv1']['b']), p['conv2']['w'])
    a2 = _get_sc_agg(32)(h2p, srcp, dstp)
    x2, h3p = _tc_epi(dp, a2, h2p, _row(p['conv2']['b']), p['conv3']['w'])
    a3 = _get_sc_agg(32)(h3p, srcp, dstp)

    pg = p['pma_g']
    hkp, hvp = _tc4(dp, a3, h3p, x1, x2, _row(p['conv3']['b']),
                    p['gmt_lin1']['w'], _row(p['gmt_lin1']['b']),
                    pg['conv_k']['w'], pg['conv_v']['w'])
    src_kv = srcp.reshape(NS, NCHUNK_KV, CH)
    dst_kv = dstp.reshape(NS, NCHUNK_KV, CH)
    akv = _get_sc_agg_kv()(hkp, hvp, src_kv, dst_kv)
    scores, V, m = _tc5(dp, akv, _row(pg['conv_k']['b']),
                        _row(pg['conv_v']['b']), pg['S'][0],
                        pg['fc_q']['w'], _row(pg['fc_q']['b']))

    batch3 = batch.astype(jnp.int32).reshape(NB, 1, NBK)
    SS = _tc6(scores, V, m, batch3)

    ps, pi = p['sab'], p['pma_i']
    logits, feat = _tc7((
        SS, pg['S'][0], pg['fc_q']['w'], _row(pg['fc_q']['b']),
        pg['fc_o']['w'], _row(pg['fc_o']['b']),
        ps['fc_q']['w'], _row(ps['fc_q']['b']),
        ps['lin_k']['w'], _row(ps['lin_k']['b']),
        ps['lin_v']['w'], _row(ps['lin_v']['b']),
        ps['fc_o']['w'], _row(ps['fc_o']['b']),
        pi['S'][0], pi['fc_q']['w'], _row(pi['fc_q']['b']),
        pi['lin_k']['w'], _row(pi['lin_k']['b']),
        pi['lin_v']['w'], _row(pi['lin_v']['b']),
        pi['fc_o']['w'], _row(pi['fc_o']['b']),
        p['gmt_lin2']['w'], _row(p['gmt_lin2']['b']),
        p['lin1']['w'], _row(p['lin1']['b']),
        p['lin2']['w'], _row(p['lin2']['b']),
    ))
    return logits, feat
